# baseline (speedup 1.0000x reference)
import functools

import jax
import jax.numpy as jnp
import numpy as np
from jax import lax
from jax.experimental import pallas as pl
from jax.experimental.pallas import tpu as pltpu
from jax.experimental.pallas import tpu_sc as plsc

N = 10000
E = 320000
F = 64
NT = 32
_ROWS = 1000

_MESH = dict(core_axis_name="c", subcore_axis_name="s")
_SC_PARAMS = pltpu.CompilerParams(needs_layout_passes=False)


def _pack_body(e_ref, sd_ref):
    sd_ref[...] = e_ref[1:2, :] * 16384 + e_ref[0:1, :]


def _pack(edge_index):
    L = 32000
    sd = pl.pallas_call(
        _pack_body,
        grid=(E // L,),
        in_specs=[pl.BlockSpec((2, L), lambda i: (0, i))],
        out_specs=pl.BlockSpec((1, L), lambda i: (0, i)),
        out_shape=jax.ShapeDtypeStruct((1, E), jnp.int32),
    )(edge_index)
    return sd.reshape(E)


def _prepare_core(x, w_ref, wl_ref, wr_ref, ft_ref, a1_ref, a2_ref, m_ref):
    ft = jnp.dot(x, w_ref[...].T, preferred_element_type=jnp.float32)
    ft_ref[...] = ft.T
    a1 = jnp.dot(ft, wl_ref[...], preferred_element_type=jnp.float32)
    a1_ref[...] = a1.T
    a2 = jnp.dot(ft, wr_ref[...], preferred_element_type=jnp.float32)
    a2_ref[...] = a2.T
    m_ref[...] = jnp.max(a2, axis=0, keepdims=True)


def _prepare_body(x_ref, w_ref, wl_ref, wr_ref, ft_ref, a1_ref, a2_ref, m_ref):
    _prepare_core(x_ref[...], w_ref, wl_ref, wr_ref, ft_ref, a1_ref, a2_ref, m_ref)


def _blockdiag(w):
    H, _, D = w.shape
    mask = jnp.asarray(np.kron(np.eye(H), np.ones((D, 1))), dtype=jnp.float32)
    return w.reshape(1, H * D).T * mask


def _prepare(last, Wfc, wl, wr):
    H, D, Din = Wfc.shape
    ft, a1, a2, a2m = pl.pallas_call(
        _prepare_body,
        out_shape=[
            jax.ShapeDtypeStruct((F, N), jnp.float32),
            jax.ShapeDtypeStruct((H, N), jnp.float32),
            jax.ShapeDtypeStruct((H, N), jnp.float32),
            jax.ShapeDtypeStruct((1, H), jnp.float32),
        ],
    )(last, Wfc.reshape(F, Din), _blockdiag(wl), _blockdiag(wr))
    return ft, a1, a2, a2m


def _phase_a(H, CH, UN):
    Q = NT // H
    EQ = E // Q
    nchunk = EQ // CH
    G = 16 * UN

    @functools.partial(
        pl.kernel,
        out_type=[
            jax.ShapeDtypeStruct((H * E,), jnp.float32),
            jax.ShapeDtypeStruct((NT * N,), jnp.float32),
        ],
        mesh=plsc.VectorSubcoreMesh(**_MESH),
        compiler_params=_SC_PARAMS,
        scratch_types=[
            pltpu.VMEM((N,), jnp.float32),
            pltpu.VMEM((N,), jnp.float32),
            pltpu.VMEM((16,), jnp.float32),
            pltpu.VMEM((N,), jnp.float32),
            [pltpu.VMEM((CH,), jnp.int32)] * 2,
            [pltpu.VMEM((CH,), jnp.float32)] * 2,
            [pltpu.SemaphoreType.DMA] * 2,
            [pltpu.SemaphoreType.DMA] * 2,
        ],
    )
    def k(sd_hbm, a1t_hbm, a2t_hbm, a2m_hbm, ee_hbm, denp_hbm,
          A1v, A2v, Mv, DENv, SDs, EEs, semi, semo):
        wid = lax.axis_index("s") * 2 + lax.axis_index("c")
        h = wid // Q
        q = wid % Q
        pltpu.sync_copy(a1t_hbm.at[h], A1v)
        pltpu.sync_copy(a2t_hbm.at[h], A2v)
        pltpu.sync_copy(a2m_hbm.at[h], Mv)

        def zero(i, _):
            DENv[pl.ds(i * 16, 16)] = jnp.zeros((16,), jnp.float32)
            return 0
        lax.fori_loop(0, N // 16, zero, 0)
        a2m = Mv[pl.ds(0, 16)]

        def start_in(kk, b):
            off = q * EQ + kk * CH
            pltpu.make_async_copy(sd_hbm.at[pl.ds(off, CH)], SDs[b], semi[b]).start()

        def wait_in(b):
            pltpu.make_async_copy(sd_hbm.at[pl.ds(0, CH)], SDs[b], semi[b]).wait()

        def wait_out(b):
            pltpu.make_async_copy(EEs[b], ee_hbm.at[pl.ds(0, CH)], semo[b]).wait()

        def compute(kk, b):
            SDv, EEv = SDs[b], EEs[b]

            @plsc.parallel_loop(0, CH // 16, unroll=UN)
            def body(i):
                ds_ = pl.ds(i * 16, 16)
                sd16 = SDv[ds_]
                s16 = sd16 & 16383
                d16 = lax.shift_right_logical(sd16, 14)
                a1d = plsc.load_gather(A1v, [d16])
                a2s = plsc.load_gather(A2v, [s16])
                t = a1d + a2s
                cm = a1d + a2m
                c = jnp.maximum(cm, cm * 0.01)
                ee = jnp.exp(jnp.maximum(t, t * 0.01) - c)
                EEv[ds_] = ee
                plsc.addupdate_scatter(DENv, [d16], ee)
            off = q * EQ + kk * CH
            pltpu.make_async_copy(EEv, ee_hbm.at[pl.ds(h * E + off, CH)], semo[b]).start()

        if nchunk >= 2:
            start_in(0, 0)

            def pair(pp, _):
                base = 2 * pp

                @pl.when(base + 1 < nchunk)
                def _():
                    start_in(base + 1, 1)
                wait_in(0)

                @pl.when(pp > 0)
                def _():
                    wait_out(0)
                compute(base, 0)

                @pl.when(base + 2 < nchunk)
                def _():
                    start_in(base + 2, 0)
                wait_in(1)

                @pl.when(pp > 0)
                def _():
                    wait_out(1)
                compute(base + 1, 1)
                return 0
            lax.fori_loop(0, nchunk // 2, pair, 0)
            wait_out(0)
            wait_out(1)
        else:
            start_in(0, 0)
            wait_in(0)
            compute(0, 0)
            wait_out(0)
        pltpu.sync_copy(DENv, denp_hbm.at[pl.ds(wid * N, N)])

    return k


def _denprep_body(H, denp_ref, den_ref):
    den_ref[...] = jnp.sum(denp_ref[...].reshape(H, NT // H, N), axis=1)


def _denprep(denp, H):
    return pl.pallas_call(
        functools.partial(_denprep_body, H),
        out_shape=jax.ShapeDtypeStruct((H, N), jnp.float32),
    )(denp)


def _phase_b(H, CH, UN):
    nchunk = E // CH
    G = 16 * UN

    @functools.partial(
        pl.kernel,
        out_type=jax.ShapeDtypeStruct((F, N), jnp.float32),
        mesh=plsc.VectorSubcoreMesh(**_MESH),
        compiler_params=_SC_PARAMS,
        scratch_types=[
            pltpu.VMEM((N,), jnp.float32),
            pltpu.VMEM((N,), jnp.float32),
            pltpu.VMEM((N,), jnp.float32),
            pltpu.VMEM((N,), jnp.float32),
            pltpu.VMEM((N,), jnp.float32),
            pltpu.VMEM((N,), jnp.float32),
            [pltpu.VMEM((CH,), jnp.int32)] * 2,
            [pltpu.VMEM((CH,), jnp.float32)] * 2,
            [pltpu.SemaphoreType.DMA] * 2,
        ],
    )
    def k(sd_hbm, ftt_hbm, ee_hbm, den_hbm, acct_hbm,
          FTa, FTb, ACCa, ACCb, DIv, TMPv, SDs, EEs, semi):
        wid = lax.axis_index("s") * 2 + lax.axis_index("c")
        h = wid // (NT // H)
        pltpu.sync_copy(ftt_hbm.at[2 * wid], FTa)
        pltpu.sync_copy(ftt_hbm.at[2 * wid + 1], FTb)
        if H == 8:
            pltpu.sync_copy(den_hbm.at[pl.ds((4 * h) * N, N)], DIv)
            for r in range(1, 4):
                pltpu.sync_copy(den_hbm.at[pl.ds((4 * h + r) * N, N)], TMPv)

                @plsc.parallel_loop(0, N // 16, unroll=5)
                def _(i):
                    ds_ = pl.ds(i * 16, 16)
                    DIv[ds_] = DIv[ds_] + TMPv[ds_]
        else:
            pltpu.sync_copy(den_hbm.at[h], DIv)

        def zero(i, _):
            z = jnp.zeros((16,), jnp.float32)
            ACCa[pl.ds(i * 16, 16)] = z
            ACCb[pl.ds(i * 16, 16)] = z
            return 0
        lax.fori_loop(0, N // 16, zero, 0)

        def start_in(kk, b):
            off = kk * CH
            pltpu.make_async_copy(sd_hbm.at[pl.ds(off, CH)], SDs[b], semi[b]).start()
            pltpu.make_async_copy(ee_hbm.at[pl.ds(h * E + off, CH)], EEs[b], semi[b]).start()

        def wait_in(b):
            pltpu.make_async_copy(sd_hbm.at[pl.ds(0, CH)], SDs[b], semi[b]).wait()
            pltpu.make_async_copy(ee_hbm.at[pl.ds(0, CH)], EEs[b], semi[b]).wait()

        def compute(b):
            SDv, EEv = SDs[b], EEs[b]

            @plsc.parallel_loop(0, CH // 16, unroll=UN)
            def body(i):
                ds_ = pl.ds(i * 16, 16)
                sd16 = SDv[ds_]
                s16 = sd16 & 16383
                d16 = lax.shift_right_logical(sd16, 14)
                w16 = EEv[ds_]
                fa = plsc.load_gather(FTa, [s16])
                plsc.addupdate_scatter(ACCa, [d16], w16 * fa)
                fb = plsc.load_gather(FTb, [s16])
                plsc.addupdate_scatter(ACCb, [d16], w16 * fb)

        start_in(0, 0)

        def pair(pp, _):
            base = 2 * pp

            @pl.when(base + 1 < nchunk)
            def _():
                start_in(base + 1, 1)
            wait_in(0)
            compute(0)

            @pl.when(base + 2 < nchunk)
            def _():
                start_in(base + 2, 0)
            wait_in(1)
            compute(1)
            return 0
        lax.fori_loop(0, nchunk // 2, pair, 0)

        @plsc.parallel_loop(0, N // 16, unroll=5)
        def norm(i):
            ds_ = pl.ds(i * 16, 16)
            den16 = DIv[ds_]
            dv = 1.0 / jnp.where(den16 > 0.0, den16, 1.0)
            ACCa[ds_] = ACCa[ds_] * dv
            ACCb[ds_] = ACCb[ds_] * dv
        pltpu.sync_copy(ACCa, acct_hbm.at[2 * wid])
        pltpu.sync_copy(ACCb, acct_hbm.at[2 * wid + 1])

    return k


def _finalize_res_body(acc_ref, last_ref, wres_ref, out_ref):
    v = acc_ref[...].T + jnp.dot(last_ref[...], wres_ref[...].T,
                                 preferred_element_type=jnp.float32)
    out_ref[...] = jnp.where(v > 0.0, v, jnp.exp(v) - 1.0)


def _finalize_nores_body(acc_ref, out_ref):
    v = acc_ref[...].T
    out_ref[...] = jnp.where(v > 0.0, v, jnp.exp(v) - 1.0)


def _finalize(accum, last, Wres):
    if Wres is None:
        return pl.pallas_call(
            _finalize_nores_body,
            out_shape=jax.ShapeDtypeStruct((N, F), jnp.float32),
        )(accum)
    H, D, Din = Wres.shape
    return pl.pallas_call(
        _finalize_res_body,
        out_shape=jax.ShapeDtypeStruct((N, F), jnp.float32),
    )(accum, last, Wres.reshape(F, Din))


def _fuse_res_body(acc_ref, last_ref, wres_ref, w_ref, wl_ref, wr_ref,
                   out_ref, ft_ref, a1_ref, a2_ref, m_ref):
    v = acc_ref[...].T + jnp.dot(last_ref[...], wres_ref[...].T,
                                 preferred_element_type=jnp.float32)
    out = jnp.where(v > 0.0, v, jnp.exp(v) - 1.0)
    out_ref[...] = out
    _prepare_core(out, w_ref, wl_ref, wr_ref, ft_ref, a1_ref, a2_ref, m_ref)


def _fuse_nores_body(acc_ref, w_ref, wl_ref, wr_ref,
                     out_ref, ft_ref, a1_ref, a2_ref, m_ref):
    v = acc_ref[...].T
    out = jnp.where(v > 0.0, v, jnp.exp(v) - 1.0)
    out_ref[...] = out
    _prepare_core(out, w_ref, wl_ref, wr_ref, ft_ref, a1_ref, a2_ref, m_ref)


def _fuse(acct, last, Wres, Wfc, wl, wr):
    Hn, Dn, Dinn = Wfc.shape
    out_shape = [
        jax.ShapeDtypeStruct((N, F), jnp.float32),
        jax.ShapeDtypeStruct((F, N), jnp.float32),
        jax.ShapeDtypeStruct((Hn, N), jnp.float32),
        jax.ShapeDtypeStruct((Hn, N), jnp.float32),
        jax.ShapeDtypeStruct((1, Hn), jnp.float32),
    ]
    wfc = Wfc.reshape(F, Dinn)
    if Wres is None:
        return pl.pallas_call(_fuse_nores_body, out_shape=out_shape)(
            acct, wfc, _blockdiag(wl), _blockdiag(wr))
    Hr, Dr, Dinr = Wres.shape
    return pl.pallas_call(_fuse_res_body, out_shape=out_shape)(
        acct, last, Wres.reshape(Hr * Dr, Dinr), wfc,
        _blockdiag(wl), _blockdiag(wr))


def _edge(H, sd, ftt, a1t, a2t, a2m):
    a2mb = jnp.broadcast_to(a2m.reshape(H, 1), (H, 16))
    if H == 8:
        ee, denp = _phase_a(H, 8000, 10)(sd, a1t, a2t, a2mb)
        return _phase_b(H, 8000, 10)(sd, ftt, ee, denp)
    ee, denp = _phase_a(H, 10000, 5)(sd, a1t, a2t, a2mb)
    den = _denprep(denp.reshape(NT, N), H)
    return _phase_b(H, 8000, 10)(sd, ftt, ee, den)


def kernel(x, edge_index, Wfc0, wl0, wr0, Wfc1, wl1, wr1, Wres1, Wfc2, wl2, wr2, Wres2):
    sd = _pack(edge_index)
    ftt, a1t, a2t, a2m = _prepare(x, Wfc0, wl0, wr0)
    acct0 = _edge(8, sd, ftt, a1t, a2t, a2m)
    out0, ftt, a1t, a2t, a2m = _fuse(acct0, None, None, Wfc1, wl1, wr1)
    acct1 = _edge(8, sd, ftt, a1t, a2t, a2m)
    out1, ftt, a1t, a2t, a2m = _fuse(acct1, out0, Wres1, Wfc2, wl2, wr2)
    acct2 = _edge(1, sd, ftt, a1t, a2t, a2m)
    return _finalize(acct2, out1, Wres2)


# speedup vs baseline: 1.0101x; 1.0101x over previous
import functools

import jax
import jax.numpy as jnp
import numpy as np
from jax import lax
from jax.experimental import pallas as pl
from jax.experimental.pallas import tpu as pltpu
from jax.experimental.pallas import tpu_sc as plsc

N = 10000
E = 320000
F = 64
NT = 32
_ROWS = 1000

_MESH = dict(core_axis_name="c", subcore_axis_name="s")
_SC_PARAMS = pltpu.CompilerParams(needs_layout_passes=False)


def _pack_body(e_ref, sd_ref):
    sd_ref[...] = e_ref[1:2, :] * 16384 + e_ref[0:1, :]


def _pack(edge_index):
    L = 32000
    sd = pl.pallas_call(
        _pack_body,
        grid=(E // L,),
        in_specs=[pl.BlockSpec((2, L), lambda i: (0, i))],
        out_specs=pl.BlockSpec((1, L), lambda i: (0, i)),
        out_shape=jax.ShapeDtypeStruct((1, E), jnp.int32),
    )(edge_index)
    return sd.reshape(E)


def _prepare_core(x, w_ref, wl_ref, wr_ref, ft_ref, a1_ref, a2_ref, m_ref):
    ft = jnp.dot(x, w_ref[...].T, preferred_element_type=jnp.float32)
    ft_ref[...] = ft.T
    a1 = jnp.dot(ft, wl_ref[...], preferred_element_type=jnp.float32)
    a1_ref[...] = a1.T
    a2 = jnp.dot(ft, wr_ref[...], preferred_element_type=jnp.float32)
    a2_ref[...] = a2.T
    m_ref[...] = jnp.max(a2, axis=0, keepdims=True)


def _prepare_body(x_ref, w_ref, wl_ref, wr_ref, ft_ref, a1_ref, a2_ref, m_ref):
    _prepare_core(x_ref[...], w_ref, wl_ref, wr_ref, ft_ref, a1_ref, a2_ref, m_ref)


def _blockdiag(w):
    H, _, D = w.shape
    mask = jnp.asarray(np.kron(np.eye(H), np.ones((D, 1))), dtype=jnp.float32)
    return w.reshape(1, H * D).T * mask


def _prepare(last, Wfc, wl, wr):
    H, D, Din = Wfc.shape
    ft, a1, a2, a2m = pl.pallas_call(
        _prepare_body,
        out_shape=[
            jax.ShapeDtypeStruct((F, N), jnp.float32),
            jax.ShapeDtypeStruct((H, N), jnp.float32),
            jax.ShapeDtypeStruct((H, N), jnp.float32),
            jax.ShapeDtypeStruct((1, H), jnp.float32),
        ],
    )(last, Wfc.reshape(F, Din), _blockdiag(wl), _blockdiag(wr))
    return ft, a1, a2, a2m


def _phase_a(H, CH, UN):
    Q = NT // H
    EQ = E // Q
    nchunk = EQ // CH
    G = 16 * UN

    @functools.partial(
        pl.kernel,
        out_type=[
            jax.ShapeDtypeStruct((H * E,), jnp.float32),
            jax.ShapeDtypeStruct((NT * N,), jnp.float32),
        ],
        mesh=plsc.VectorSubcoreMesh(**_MESH),
        compiler_params=_SC_PARAMS,
        scratch_types=[
            pltpu.VMEM((N,), jnp.float32),
            pltpu.VMEM((N,), jnp.float32),
            pltpu.VMEM((16,), jnp.float32),
            pltpu.VMEM((N,), jnp.float32),
            [pltpu.VMEM((CH,), jnp.int32)] * 2,
            [pltpu.VMEM((CH,), jnp.float32)] * 2,
            [pltpu.SemaphoreType.DMA] * 2,
            [pltpu.SemaphoreType.DMA] * 2,
        ],
    )
    def k(sd_hbm, a1t_hbm, a2t_hbm, a2m_hbm, ee_hbm, denp_hbm,
          A1v, A2v, Mv, DENv, SDs, EEs, semi, semo):
        wid = lax.axis_index("s") * 2 + lax.axis_index("c")
        h = wid // Q
        q = wid % Q
        pltpu.sync_copy(a1t_hbm.at[h], A1v)
        pltpu.sync_copy(a2t_hbm.at[h], A2v)
        pltpu.sync_copy(a2m_hbm.at[h], Mv)

        def zero(i, _):
            DENv[pl.ds(i * 16, 16)] = jnp.zeros((16,), jnp.float32)
            return 0
        lax.fori_loop(0, N // 16, zero, 0)
        a2m = Mv[pl.ds(0, 16)]

        def start_in(kk, b):
            off = q * EQ + kk * CH
            pltpu.make_async_copy(sd_hbm.at[pl.ds(off, CH)], SDs[b], semi[b]).start()

        def wait_in(b):
            pltpu.make_async_copy(sd_hbm.at[pl.ds(0, CH)], SDs[b], semi[b]).wait()

        def wait_out(b):
            pltpu.make_async_copy(EEs[b], ee_hbm.at[pl.ds(0, CH)], semo[b]).wait()

        def compute(kk, b):
            SDv, EEv = SDs[b], EEs[b]

            @plsc.parallel_loop(0, CH // 16, unroll=UN)
            def body(i):
                ds_ = pl.ds(i * 16, 16)
                sd16 = SDv[ds_]
                s16 = sd16 & 16383
                d16 = lax.shift_right_logical(sd16, 14)
                a1d = plsc.load_gather(A1v, [d16])
                a2s = plsc.load_gather(A2v, [s16])
                t = a1d + a2s
                cm = a1d + a2m
                c = jnp.maximum(cm, cm * 0.01)
                ee = jnp.exp(jnp.maximum(t, t * 0.01) - c)
                EEv[ds_] = ee
                plsc.addupdate_scatter(DENv, [d16], ee)
            off = q * EQ + kk * CH
            pltpu.make_async_copy(EEv, ee_hbm.at[pl.ds(h * E + off, CH)], semo[b]).start()

        if nchunk >= 2:
            start_in(0, 0)

            def pair(pp, _):
                base = 2 * pp

                @pl.when(base + 1 < nchunk)
                def _():
                    start_in(base + 1, 1)
                wait_in(0)

                @pl.when(pp > 0)
                def _():
                    wait_out(0)
                compute(base, 0)

                @pl.when(base + 2 < nchunk)
                def _():
                    start_in(base + 2, 0)
                wait_in(1)

                @pl.when(pp > 0)
                def _():
                    wait_out(1)
                compute(base + 1, 1)
                return 0
            lax.fori_loop(0, nchunk // 2, pair, 0)
            wait_out(0)
            wait_out(1)
        else:
            start_in(0, 0)
            wait_in(0)
            compute(0, 0)
            wait_out(0)
        pltpu.sync_copy(DENv, denp_hbm.at[pl.ds(wid * N, N)])

    return k


def _denprep_body(H, denp_ref, den_ref):
    den_ref[...] = jnp.sum(denp_ref[...].reshape(H, NT // H, N), axis=1)


def _denprep(denp, H):
    return pl.pallas_call(
        functools.partial(_denprep_body, H),
        out_shape=jax.ShapeDtypeStruct((H, N), jnp.float32),
    )(denp)


def _phase_b(H, CH, UN):
    nchunk = E // CH
    G = 16 * UN

    @functools.partial(
        pl.kernel,
        out_type=jax.ShapeDtypeStruct((F, N), jnp.float32),
        mesh=plsc.VectorSubcoreMesh(**_MESH),
        compiler_params=_SC_PARAMS,
        scratch_types=[
            pltpu.VMEM((N,), jnp.float32),
            pltpu.VMEM((N,), jnp.float32),
            pltpu.VMEM((N,), jnp.float32),
            pltpu.VMEM((N,), jnp.float32),
            pltpu.VMEM((N,), jnp.float32),
            pltpu.VMEM((N,), jnp.float32),
            [pltpu.VMEM((CH,), jnp.int32)] * 2,
            [pltpu.VMEM((CH,), jnp.float32)] * 2,
            [pltpu.SemaphoreType.DMA] * 2,
        ],
    )
    def k(sd_hbm, ftt_hbm, ee_hbm, den_hbm, acct_hbm,
          FTa, FTb, ACCa, ACCb, DIv, TMPv, SDs, EEs, semi):
        wid = lax.axis_index("s") * 2 + lax.axis_index("c")
        h = wid // (NT // H)
        pltpu.sync_copy(ftt_hbm.at[2 * wid], FTa)
        pltpu.sync_copy(ftt_hbm.at[2 * wid + 1], FTb)
        if H == 8:
            pltpu.sync_copy(den_hbm.at[pl.ds((4 * h) * N, N)], DIv)
            for r in range(1, 4):
                pltpu.sync_copy(den_hbm.at[pl.ds((4 * h + r) * N, N)], TMPv)

                @plsc.parallel_loop(0, N // 16, unroll=5)
                def _(i):
                    ds_ = pl.ds(i * 16, 16)
                    DIv[ds_] = DIv[ds_] + TMPv[ds_]
        else:
            pltpu.sync_copy(den_hbm.at[h], DIv)

        def zero(i, _):
            z = jnp.zeros((16,), jnp.float32)
            ACCa[pl.ds(i * 16, 16)] = z
            ACCb[pl.ds(i * 16, 16)] = z
            return 0
        lax.fori_loop(0, N // 16, zero, 0)

        def start_in(kk, b):
            off = kk * CH
            pltpu.make_async_copy(sd_hbm.at[pl.ds(off, CH)], SDs[b], semi[b]).start()
            pltpu.make_async_copy(ee_hbm.at[pl.ds(h * E + off, CH)], EEs[b], semi[b]).start()

        def wait_in(b):
            pltpu.make_async_copy(sd_hbm.at[pl.ds(0, CH)], SDs[b], semi[b]).wait()
            pltpu.make_async_copy(ee_hbm.at[pl.ds(0, CH)], EEs[b], semi[b]).wait()

        def compute(b):
            SDv, EEv = SDs[b], EEs[b]

            @plsc.parallel_loop(0, CH // 16, unroll=UN)
            def body(i):
                ds_ = pl.ds(i * 16, 16)
                sd16 = SDv[ds_]
                s16 = sd16 & 16383
                d16 = lax.shift_right_logical(sd16, 14)
                w16 = EEv[ds_]
                fa = plsc.load_gather(FTa, [s16])
                plsc.addupdate_scatter(ACCa, [d16], w16 * fa)
                fb = plsc.load_gather(FTb, [s16])
                plsc.addupdate_scatter(ACCb, [d16], w16 * fb)

        start_in(0, 0)

        def pair(pp, _):
            base = 2 * pp

            @pl.when(base + 1 < nchunk)
            def _():
                start_in(base + 1, 1)
            wait_in(0)
            compute(0)

            @pl.when(base + 2 < nchunk)
            def _():
                start_in(base + 2, 0)
            wait_in(1)
            compute(1)
            return 0
        lax.fori_loop(0, nchunk // 2, pair, 0)

        @plsc.parallel_loop(0, N // 16, unroll=5)
        def norm(i):
            ds_ = pl.ds(i * 16, 16)
            den16 = DIv[ds_]
            dv = 1.0 / jnp.where(den16 > 0.0, den16, 1.0)
            ACCa[ds_] = ACCa[ds_] * dv
            ACCb[ds_] = ACCb[ds_] * dv
        pltpu.sync_copy(ACCa, acct_hbm.at[2 * wid])
        pltpu.sync_copy(ACCb, acct_hbm.at[2 * wid + 1])

    return k


def _finalize_res_body(acc_ref, last_ref, wres_ref, out_ref):
    v = acc_ref[...].T + jnp.dot(last_ref[...], wres_ref[...].T,
                                 preferred_element_type=jnp.float32)
    out_ref[...] = jnp.where(v > 0.0, v, jnp.exp(v) - 1.0)


def _finalize_nores_body(acc_ref, out_ref):
    v = acc_ref[...].T
    out_ref[...] = jnp.where(v > 0.0, v, jnp.exp(v) - 1.0)


def _finalize(accum, last, Wres):
    if Wres is None:
        return pl.pallas_call(
            _finalize_nores_body,
            out_shape=jax.ShapeDtypeStruct((N, F), jnp.float32),
        )(accum)
    H, D, Din = Wres.shape
    return pl.pallas_call(
        _finalize_res_body,
        out_shape=jax.ShapeDtypeStruct((N, F), jnp.float32),
    )(accum, last, Wres.reshape(F, Din))


def _fuse_res_body(acc_ref, last_ref, wres_ref, w_ref, wl_ref, wr_ref,
                   out_ref, ft_ref, a1_ref, a2_ref, m_ref):
    v = acc_ref[...].T + jnp.dot(last_ref[...], wres_ref[...].T,
                                 preferred_element_type=jnp.float32)
    out = jnp.where(v > 0.0, v, jnp.exp(v) - 1.0)
    out_ref[...] = out
    _prepare_core(out, w_ref, wl_ref, wr_ref, ft_ref, a1_ref, a2_ref, m_ref)


def _fuse_nores_body(acc_ref, w_ref, wl_ref, wr_ref,
                     out_ref, ft_ref, a1_ref, a2_ref, m_ref):
    v = acc_ref[...].T
    out = jnp.where(v > 0.0, v, jnp.exp(v) - 1.0)
    out_ref[...] = out
    _prepare_core(out, w_ref, wl_ref, wr_ref, ft_ref, a1_ref, a2_ref, m_ref)


def _fuse(acct, last, Wres, Wfc, wl, wr):
    Hn, Dn, Dinn = Wfc.shape
    out_shape = [
        jax.ShapeDtypeStruct((N, F), jnp.float32),
        jax.ShapeDtypeStruct((F, N), jnp.float32),
        jax.ShapeDtypeStruct((Hn, N), jnp.float32),
        jax.ShapeDtypeStruct((Hn, N), jnp.float32),
        jax.ShapeDtypeStruct((1, Hn), jnp.float32),
    ]
    wfc = Wfc.reshape(F, Dinn)
    if Wres is None:
        return pl.pallas_call(_fuse_nores_body, out_shape=out_shape)(
            acct, wfc, _blockdiag(wl), _blockdiag(wr))
    Hr, Dr, Dinr = Wres.shape
    return pl.pallas_call(_fuse_res_body, out_shape=out_shape)(
        acct, last, Wres.reshape(Hr * Dr, Dinr), wfc,
        _blockdiag(wl), _blockdiag(wr))


def _edge(H, sd, ftt, a1t, a2t, a2m):
    a2mb = jnp.broadcast_to(a2m.reshape(H, 1), (H, 16))
    if H == 8:
        ee, denp = _phase_a(H, 8000, 5)(sd, a1t, a2t, a2mb)
        return _phase_b(H, 8000, 5)(sd, ftt, ee, denp)
    ee, denp = _phase_a(H, 10000, 5)(sd, a1t, a2t, a2mb)
    den = _denprep(denp.reshape(NT, N), H)
    return _phase_b(H, 8000, 5)(sd, ftt, ee, den)


def kernel(x, edge_index, Wfc0, wl0, wr0, Wfc1, wl1, wr1, Wres1, Wfc2, wl2, wr2, Wres2):
    sd = _pack(edge_index)
    ftt, a1t, a2t, a2m = _prepare(x, Wfc0, wl0, wr0)
    acct0 = _edge(8, sd, ftt, a1t, a2t, a2m)
    out0, ftt, a1t, a2t, a2m = _fuse(acct0, None, None, Wfc1, wl1, wr1)
    acct1 = _edge(8, sd, ftt, a1t, a2t, a2m)
    out1, ftt, a1t, a2t, a2m = _fuse(acct1, out0, Wres1, Wfc2, wl2, wr2)
    acct2 = _edge(1, sd, ftt, a1t, a2t, a2m)
    return _finalize(acct2, out1, Wres2)
